# baseline (speedup 1.0000x reference)
import functools

import jax
import jax.numpy as jnp
from jax import lax
from jax.experimental import pallas as pl
from jax.experimental.pallas import tpu as pltpu
from jax.experimental.pallas import tpu_sc as plsc

B, S, H = 2, 4096, 768
E, TOPK, FF = 16, 2, 384
SFF = 384 * 2
T = B * S
N = T * TOPK
TM = 256
TG = 256
NPAD = N + E * TG
NT = NPAD // TG


def _gate_shared_body(x_ref, gw_ref, s1_ref, s2_ref, s3_ref,
                      idx_ref, w_ref, sh_ref):
    x = x_ref[...]
    logits = lax.dot_general(x, gw_ref[...], (((1,), (1,)), ((), ())),
                             preferred_element_type=jnp.float32)
    m = jnp.max(logits, axis=-1, keepdims=True)
    p = jnp.exp(logits - m)
    s = p / jnp.sum(p, axis=-1, keepdims=True)
    iota = lax.broadcasted_iota(jnp.int32, (TM, E), 1)
    m1 = jnp.max(s, axis=-1, keepdims=True)
    i1 = jnp.min(jnp.where(s == m1, iota, E), axis=-1, keepdims=True)
    oh1 = iota == i1
    s2 = jnp.where(oh1, -1.0, s)
    m2 = jnp.max(s2, axis=-1, keepdims=True)
    i2 = jnp.min(jnp.where(s2 == m2, iota, E), axis=-1, keepdims=True)
    denom = m1 + m2 + 1e-20
    idx_ref[...] = jnp.concatenate([i1, i2], axis=1)
    w_ref[...] = jnp.concatenate([m1 / denom, m2 / denom], axis=1)
    xb = x.astype(jnp.bfloat16)
    g = lax.dot_general(xb, s1_ref[...], (((1,), (1,)), ((), ())),
                        preferred_element_type=jnp.float32)
    u = lax.dot_general(xb, s2_ref[...], (((1,), (1,)), ((), ())),
                        preferred_element_type=jnp.float32)
    a = ((g * jax.nn.sigmoid(g)) * u).astype(jnp.bfloat16)
    sh_ref[...] = lax.dot_general(
        a, s3_ref[...], (((1,), (0,)), ((), ())),
        preferred_element_type=jnp.float32).astype(jnp.bfloat16)


def _gate_shared(x, gw, s1, s2, s3):
    return pl.pallas_call(
        _gate_shared_body,
        grid=(T // TM,),
        in_specs=[
            pl.BlockSpec((TM, H), lambda i: (i, 0)),
            pl.BlockSpec((E, H), lambda i: (0, 0)),
            pl.BlockSpec((SFF, H), lambda i: (0, 0)),
            pl.BlockSpec((SFF, H), lambda i: (0, 0)),
            pl.BlockSpec((SFF, H), lambda i: (0, 0)),
        ],
        out_specs=[
            pl.BlockSpec((TM, TOPK), lambda i: (i, 0)),
            pl.BlockSpec((TM, TOPK), lambda i: (i, 0)),
            pl.BlockSpec((TM, H), lambda i: (i, 0)),
        ],
        out_shape=[
            jax.ShapeDtypeStruct((T, TOPK), jnp.int32),
            jax.ShapeDtypeStruct((T, TOPK), jnp.float32),
            jax.ShapeDtypeStruct((T, H), jnp.bfloat16),
        ],
    )(x, gw, s1, s2, s3)


def _grouped_mlp_body(eid_ref, xs_ref, w_ref, wg_ref, wu_ref, wd_ref,
                      ys_ref):
    xb = xs_ref[...]
    g = lax.dot_general(xb, wg_ref[0], (((1,), (1,)), ((), ())),
                        preferred_element_type=jnp.float32)
    u = lax.dot_general(xb, wu_ref[0], (((1,), (1,)), ((), ())),
                        preferred_element_type=jnp.float32)
    a = ((g * jax.nn.sigmoid(g)) * u * w_ref[...]).astype(jnp.bfloat16)
    ys_ref[...] = lax.dot_general(
        a, wd_ref[0], (((1,), (1,)), ((), ())),
        preferred_element_type=jnp.float32).astype(jnp.bfloat16)


def _grouped_mlp(tile_eid, xs, sorted_w, wg, wu, wd):
    grid_spec = pltpu.PrefetchScalarGridSpec(
        num_scalar_prefetch=1,
        grid=(NT,),
        in_specs=[
            pl.BlockSpec((TG, H), lambda i, eid: (i, 0)),
            pl.BlockSpec((TG, 1), lambda i, eid: (i, 0)),
            pl.BlockSpec((1, FF, H), lambda i, eid: (eid[i], 0, 0)),
            pl.BlockSpec((1, FF, H), lambda i, eid: (eid[i], 0, 0)),
            pl.BlockSpec((1, H, FF), lambda i, eid: (eid[i], 0, 0)),
        ],
        out_specs=pl.BlockSpec((TG, H), lambda i, eid: (i, 0)),
    )
    return pl.pallas_call(
        _grouped_mlp_body,
        grid_spec=grid_spec,
        out_shape=jax.ShapeDtypeStruct((NPAD, H), jnp.bfloat16),
    )(tile_eid, xs, sorted_w, wg, wu, wd)


SC_NC, SC_NS = 2, 16
NW = SC_NC * SC_NS
GR = 128
HI = H // 2


@functools.lru_cache(maxsize=None)
def _sc_mesh():
    return plsc.VectorSubcoreMesh(core_axis_name="c", subcore_axis_name="s")


@functools.lru_cache(maxsize=None)
def _sc_gather_kernel(n_out):
    def body(tab_hbm, idx_hbm, out_hbm, idx_v, rows_v, sem):
        wid = lax.axis_index("s") * SC_NC + lax.axis_index("c")
        rows_per_w = n_out // NW
        base_w = wid * rows_per_w

        def chunk(i, _):
            base = base_w + i * GR
            pltpu.sync_copy(idx_hbm.at[pl.ds(base, GR)], idx_v)
            pltpu.async_copy(tab_hbm.at[idx_v], rows_v, sem).wait()
            pltpu.sync_copy(rows_v, out_hbm.at[pl.ds(base, GR)])
            return _

        lax.fori_loop(0, rows_per_w // GR, chunk, 0)

    return pl.kernel(
        body,
        out_type=jax.ShapeDtypeStruct((n_out, HI), jnp.int32),
        mesh=_sc_mesh(),
        scratch_types=[
            pltpu.VMEM((GR,), jnp.int32),
            pltpu.VMEM((GR, HI), jnp.int32),
            pltpu.SemaphoreType.DMA,
        ],
    )


def _combine_body(g01_ref, g1_ref, sh_ref, y_ref):
    y_ref[...] = (g01_ref[...].astype(jnp.float32)
                  + g1_ref[...].astype(jnp.float32)
                  + sh_ref[...].astype(jnp.float32))


def _combine(g01, sh):
    nb = T // TM
    return pl.pallas_call(
        _combine_body,
        grid=(nb,),
        in_specs=[
            pl.BlockSpec((TM, H), lambda i: (i, 0)),
            pl.BlockSpec((TM, H), lambda i, nb=nb: (i + nb, 0)),
            pl.BlockSpec((TM, H), lambda i: (i, 0)),
        ],
        out_specs=pl.BlockSpec((TM, H), lambda i: (i, 0)),
        out_shape=jax.ShapeDtypeStruct((T, H), jnp.float32),
    )(g01, g01, sh)


def _route(idx, wts):
    eid = idx.reshape(N)
    wf = wts.reshape(N)
    CH = 128
    oh = (eid[:, None] == jnp.arange(E, dtype=jnp.int32)[None, :]
          ).astype(jnp.float32).reshape(N // CH, CH, E)
    tril = jnp.tril(jnp.ones((CH, CH), jnp.float32))
    within = jnp.einsum('rc,bce->bre', tril, oh,
                        preferred_element_type=jnp.float32)
    blk_tot = within[:, -1, :]
    blk_pre = jnp.cumsum(blk_tot, axis=0) - blk_tot
    ranks = (within + blk_pre[:, None, :]).reshape(N, E)
    counts = (blk_pre[-1] + blk_tot[-1]).astype(jnp.int32)
    padded = ((counts + TG - 1) // TG) * TG
    pcum = jnp.cumsum(padded)
    base = pcum - padded
    rank_n = jnp.take_along_axis(ranks, eid[:, None], axis=1)[:, 0]
    pos = base[eid] + rank_n.astype(jnp.int32) - 1
    sorted_tok = jnp.zeros((NPAD,), jnp.int32).at[pos].set(
        jnp.arange(N, dtype=jnp.int32) // TOPK)
    sorted_w = jnp.zeros((NPAD,), jnp.float32).at[pos].set(wf)
    tile_start = jnp.arange(NT, dtype=jnp.int32) * TG
    tile_eid = jnp.minimum(
        jnp.sum((tile_start[:, None] >= pcum[None, :]).astype(jnp.int32),
                axis=1), E - 1).astype(jnp.int32)
    return pos, sorted_tok, sorted_w, tile_eid


@jax.jit
def _moe(x, gw, wg, wu, wd, s1, s2, s3):
    idx, wts, shared = _gate_shared(x, gw, s1, s2, s3)
    pos, sorted_tok, sorted_w, tile_eid = _route(idx, wts)
    x_i = lax.bitcast_convert_type(
        x.astype(jnp.bfloat16).reshape(T, HI, 2), jnp.int32)
    xs_i = _sc_gather_kernel(NPAD)(x_i, sorted_tok)
    xs = lax.bitcast_convert_type(xs_i, jnp.bfloat16).reshape(NPAD, H)
    ys = _grouped_mlp(tile_eid, xs, sorted_w[:, None], wg, wu, wd)
    ys_i = lax.bitcast_convert_type(
        ys.reshape(NPAD, HI, 2), jnp.int32)
    pos2 = pos.reshape(T, TOPK)
    pos_cat = jnp.concatenate([pos2[:, 0], pos2[:, 1]])
    g01_i = _sc_gather_kernel(N)(ys_i, pos_cat)
    g01 = lax.bitcast_convert_type(g01_i, jnp.bfloat16).reshape(N, H)
    return _combine(g01, shared)


def kernel(hidden_states, gate_weight, Wg, Wu, Wd, sWg, sWu, sWd):
    x = hidden_states.reshape(T, H)
    wg = Wg.astype(jnp.bfloat16)
    wu = Wu.astype(jnp.bfloat16)
    wd = Wd.astype(jnp.bfloat16)
    s1 = sWg.astype(jnp.bfloat16)
    s2 = sWu.astype(jnp.bfloat16)
    s3 = sWd.T.astype(jnp.bfloat16)
    y = _moe(x, gate_weight, wg, wu, wd, s1, s2, s3)
    return y.reshape(B, S, H)


# speedup vs baseline: 2.4795x; 2.4795x over previous
import functools

import jax
import jax.numpy as jnp
from jax import lax
from jax.experimental import pallas as pl
from jax.experimental.pallas import tpu as pltpu
from jax.experimental.pallas import tpu_sc as plsc

B, S, H = 2, 4096, 768
E, TOPK, FF = 16, 2, 384
SFF = 384 * 2
T = B * S
N = T * TOPK
TM = 256
TG = 256
NPAD = N + E * TG
NT = NPAD // TG


def _gate_shared_body(x_ref, gw_ref, s1_ref, s2_ref, s3_ref,
                      idx_ref, w_ref, sh_ref):
    x = x_ref[...]
    logits = lax.dot_general(x, gw_ref[...], (((1,), (1,)), ((), ())),
                             preferred_element_type=jnp.float32)
    m = jnp.max(logits, axis=-1, keepdims=True)
    p = jnp.exp(logits - m)
    s = p / jnp.sum(p, axis=-1, keepdims=True)
    iota = lax.broadcasted_iota(jnp.int32, (TM, E), 1)
    m1 = jnp.max(s, axis=-1, keepdims=True)
    i1 = jnp.min(jnp.where(s == m1, iota, E), axis=-1, keepdims=True)
    oh1 = iota == i1
    s2 = jnp.where(oh1, -1.0, s)
    m2 = jnp.max(s2, axis=-1, keepdims=True)
    i2 = jnp.min(jnp.where(s2 == m2, iota, E), axis=-1, keepdims=True)
    denom = m1 + m2 + 1e-20
    idx_ref[...] = jnp.concatenate([i1, i2], axis=1)
    w_ref[...] = jnp.concatenate([m1 / denom, m2 / denom], axis=1)
    xb = x.astype(jnp.bfloat16)
    g = lax.dot_general(xb, s1_ref[...], (((1,), (1,)), ((), ())),
                        preferred_element_type=jnp.float32)
    u = lax.dot_general(xb, s2_ref[...], (((1,), (1,)), ((), ())),
                        preferred_element_type=jnp.float32)
    a = ((g * jax.nn.sigmoid(g)) * u).astype(jnp.bfloat16)
    sh_ref[...] = lax.dot_general(a, s3_ref[...], (((1,), (0,)), ((), ())),
                                  preferred_element_type=jnp.float32)


def _gate_shared(x, gw, s1, s2, s3):
    return pl.pallas_call(
        _gate_shared_body,
        grid=(T // TM,),
        in_specs=[
            pl.BlockSpec((TM, H), lambda i: (i, 0)),
            pl.BlockSpec((E, H), lambda i: (0, 0)),
            pl.BlockSpec((SFF, H), lambda i: (0, 0)),
            pl.BlockSpec((SFF, H), lambda i: (0, 0)),
            pl.BlockSpec((SFF, H), lambda i: (0, 0)),
        ],
        out_specs=[
            pl.BlockSpec((TM, TOPK), lambda i: (i, 0)),
            pl.BlockSpec((TM, TOPK), lambda i: (i, 0)),
            pl.BlockSpec((TM, H), lambda i: (i, 0)),
        ],
        out_shape=[
            jax.ShapeDtypeStruct((T, TOPK), jnp.int32),
            jax.ShapeDtypeStruct((T, TOPK), jnp.float32),
            jax.ShapeDtypeStruct((T, H), jnp.float32),
        ],
    )(x, gw, s1, s2, s3)


def _grouped_mlp_body(eid_ref, xs_ref, w_ref, wg_ref, wu_ref, wd_ref,
                      ys_ref):
    xb = xs_ref[...].astype(jnp.bfloat16)
    g = lax.dot_general(xb, wg_ref[0], (((1,), (1,)), ((), ())),
                        preferred_element_type=jnp.float32)
    u = lax.dot_general(xb, wu_ref[0], (((1,), (1,)), ((), ())),
                        preferred_element_type=jnp.float32)
    a = ((g * jax.nn.sigmoid(g)) * u * w_ref[...]).astype(jnp.bfloat16)
    ys_ref[...] = lax.dot_general(a, wd_ref[0], (((1,), (1,)), ((), ())),
                                  preferred_element_type=jnp.float32)


def _grouped_mlp(tile_eid, xs, sorted_w, wg, wu, wd):
    grid_spec = pltpu.PrefetchScalarGridSpec(
        num_scalar_prefetch=1,
        grid=(NT,),
        in_specs=[
            pl.BlockSpec((TG, H), lambda i, eid: (i, 0)),
            pl.BlockSpec((TG, 1), lambda i, eid: (i, 0)),
            pl.BlockSpec((1, FF, H), lambda i, eid: (eid[i], 0, 0)),
            pl.BlockSpec((1, FF, H), lambda i, eid: (eid[i], 0, 0)),
            pl.BlockSpec((1, H, FF), lambda i, eid: (eid[i], 0, 0)),
        ],
        out_specs=pl.BlockSpec((TG, H), lambda i, eid: (i, 0)),
    )
    return pl.pallas_call(
        _grouped_mlp_body,
        grid_spec=grid_spec,
        out_shape=jax.ShapeDtypeStruct((NPAD, H), jnp.float32),
    )(tile_eid, xs, sorted_w, wg, wu, wd)


SC_NC, SC_NS = 2, 16
NW = SC_NC * SC_NS
GR = 64


@functools.lru_cache(maxsize=None)
def _sc_mesh():
    return plsc.VectorSubcoreMesh(core_axis_name="c", subcore_axis_name="s")


@functools.lru_cache(maxsize=None)
def _sc_gather_kernel(n_out):
    def body(tab_hbm, idx_hbm, out_hbm, idx_v, rows_v, sem):
        wid = lax.axis_index("s") * SC_NC + lax.axis_index("c")
        rows_per_w = n_out // NW
        base_w = wid * rows_per_w

        def chunk(i, _):
            base = base_w + i * GR
            pltpu.sync_copy(idx_hbm.at[pl.ds(base, GR)], idx_v)
            pltpu.async_copy(tab_hbm.at[idx_v], rows_v, sem).wait()
            pltpu.sync_copy(rows_v, out_hbm.at[pl.ds(base, GR)])
            return _

        lax.fori_loop(0, rows_per_w // GR, chunk, 0)

    return pl.kernel(
        body,
        out_type=jax.ShapeDtypeStruct((n_out, H), jnp.float32),
        mesh=_sc_mesh(),
        scratch_types=[
            pltpu.VMEM((GR,), jnp.int32),
            pltpu.VMEM((GR, H), jnp.float32),
            pltpu.SemaphoreType.DMA,
        ],
    )


def _combine_body(g01_ref, g1_ref, sh_ref, y_ref):
    y_ref[...] = (g01_ref[...].astype(jnp.float32)
                  + g1_ref[...].astype(jnp.float32)
                  + sh_ref[...].astype(jnp.float32))


def _combine(g01, sh):
    nb = T // TM
    return pl.pallas_call(
        _combine_body,
        grid=(nb,),
        in_specs=[
            pl.BlockSpec((TM, H), lambda i: (i, 0)),
            pl.BlockSpec((TM, H), lambda i, nb=nb: (i + nb, 0)),
            pl.BlockSpec((TM, H), lambda i: (i, 0)),
        ],
        out_specs=pl.BlockSpec((TM, H), lambda i: (i, 0)),
        out_shape=jax.ShapeDtypeStruct((T, H), jnp.float32),
    )(g01, g01, sh)


def _route(idx, wts):
    eid = idx.reshape(N)
    wf = wts.reshape(N)
    CH = 128
    oh = (eid[:, None] == jnp.arange(E, dtype=jnp.int32)[None, :]
          ).astype(jnp.float32).reshape(N // CH, CH, E)
    tril = jnp.tril(jnp.ones((CH, CH), jnp.float32))
    within = jnp.einsum('rc,bce->bre', tril, oh,
                        preferred_element_type=jnp.float32)
    blk_tot = within[:, -1, :]
    blk_pre = jnp.cumsum(blk_tot, axis=0) - blk_tot
    ranks = (within + blk_pre[:, None, :]).reshape(N, E)
    counts = (blk_pre[-1] + blk_tot[-1]).astype(jnp.int32)
    padded = ((counts + TG - 1) // TG) * TG
    pcum = jnp.cumsum(padded)
    base = pcum - padded
    rank_n = jnp.take_along_axis(ranks, eid[:, None], axis=1)[:, 0]
    pos = base[eid] + rank_n.astype(jnp.int32) - 1
    sorted_tok = jnp.zeros((NPAD,), jnp.int32).at[pos].set(
        jnp.arange(N, dtype=jnp.int32) // TOPK)
    sorted_w = jnp.zeros((NPAD,), jnp.float32).at[pos].set(wf)
    tile_start = jnp.arange(NT, dtype=jnp.int32) * TG
    tile_eid = jnp.minimum(
        jnp.sum((tile_start[:, None] >= pcum[None, :]).astype(jnp.int32),
                axis=1), E - 1).astype(jnp.int32)
    return pos, sorted_tok, sorted_w, tile_eid


@jax.jit
def _moe(x, gw, wg, wu, wd, s1, s2, s3):
    idx, wts, shared = _gate_shared(x, gw, s1, s2, s3)
    pos, sorted_tok, sorted_w, tile_eid = _route(idx, wts)
    xs = _sc_gather_kernel(NPAD)(x, sorted_tok)
    ys = _grouped_mlp(tile_eid, xs, sorted_w[:, None], wg, wu, wd)
    pos2 = pos.reshape(T, TOPK)
    pos_cat = jnp.concatenate([pos2[:, 0], pos2[:, 1]])
    g01 = _sc_gather_kernel(N)(ys, pos_cat)
    return _combine(g01, shared)


def kernel(hidden_states, gate_weight, Wg, Wu, Wd, sWg, sWu, sWd):
    x = hidden_states.reshape(T, H)
    wg = Wg.astype(jnp.bfloat16)
    wu = Wu.astype(jnp.bfloat16)
    wd = Wd.astype(jnp.bfloat16)
    s1 = sWg.astype(jnp.bfloat16)
    s2 = sWu.astype(jnp.bfloat16)
    s3 = sWd.T.astype(jnp.bfloat16)
    y = _moe(x, gate_weight, wg, wu, wd, s1, s2, s3)
    return y.reshape(B, S, H)


# speedup vs baseline: 3.3026x; 1.3320x over previous
import functools

import jax
import jax.numpy as jnp
from jax import lax
from jax.experimental import pallas as pl
from jax.experimental.pallas import tpu as pltpu
from jax.experimental.pallas import tpu_sc as plsc

B, S, H = 2, 4096, 768
E, TOPK, FF = 16, 2, 384
SFF = 384 * 2
T = B * S
N = T * TOPK
TM = 256
TG = 256
NPAD = N + E * TG
NT = NPAD // TG


def _gate_shared_body(x_ref, gw_ref, s1_ref, s2_ref, s3_ref,
                      idx_ref, w_ref, sh_ref):
    x = x_ref[...]
    logits = lax.dot_general(x, gw_ref[...], (((1,), (1,)), ((), ())),
                             preferred_element_type=jnp.float32)
    m = jnp.max(logits, axis=-1, keepdims=True)
    p = jnp.exp(logits - m)
    s = p / jnp.sum(p, axis=-1, keepdims=True)
    iota = lax.broadcasted_iota(jnp.int32, (TM, E), 1)
    m1 = jnp.max(s, axis=-1, keepdims=True)
    i1 = jnp.min(jnp.where(s == m1, iota, E), axis=-1, keepdims=True)
    oh1 = iota == i1
    s2 = jnp.where(oh1, -1.0, s)
    m2 = jnp.max(s2, axis=-1, keepdims=True)
    i2 = jnp.min(jnp.where(s2 == m2, iota, E), axis=-1, keepdims=True)
    denom = m1 + m2 + 1e-20
    idx_ref[...] = jnp.concatenate([i1, i2], axis=1)
    w_ref[...] = jnp.concatenate([m1 / denom, m2 / denom], axis=1)
    xb = x.astype(jnp.bfloat16)
    g = lax.dot_general(xb, s1_ref[...], (((1,), (1,)), ((), ())),
                        preferred_element_type=jnp.float32)
    u = lax.dot_general(xb, s2_ref[...], (((1,), (1,)), ((), ())),
                        preferred_element_type=jnp.float32)
    a = ((g * jax.nn.sigmoid(g)) * u).astype(jnp.bfloat16)
    sh_ref[...] = lax.dot_general(a, s3_ref[...], (((1,), (0,)), ((), ())),
                                  preferred_element_type=jnp.float32)


def _gate_shared(x, gw, s1, s2, s3):
    return pl.pallas_call(
        _gate_shared_body,
        grid=(T // TM,),
        in_specs=[
            pl.BlockSpec((TM, H), lambda i: (i, 0)),
            pl.BlockSpec((E, H), lambda i: (0, 0)),
            pl.BlockSpec((SFF, H), lambda i: (0, 0)),
            pl.BlockSpec((SFF, H), lambda i: (0, 0)),
            pl.BlockSpec((SFF, H), lambda i: (0, 0)),
        ],
        out_specs=[
            pl.BlockSpec((TM, TOPK), lambda i: (i, 0)),
            pl.BlockSpec((TM, TOPK), lambda i: (i, 0)),
            pl.BlockSpec((TM, H), lambda i: (i, 0)),
        ],
        out_shape=[
            jax.ShapeDtypeStruct((T, TOPK), jnp.int32),
            jax.ShapeDtypeStruct((T, TOPK), jnp.float32),
            jax.ShapeDtypeStruct((T, H), jnp.float32),
        ],
    )(x, gw, s1, s2, s3)


def _grouped_mlp_body(eid_ref, xs_ref, w_ref, wg_ref, wu_ref, wd_ref,
                      ys_ref):
    xb = xs_ref[...].astype(jnp.bfloat16)
    g = lax.dot_general(xb, wg_ref[0], (((1,), (1,)), ((), ())),
                        preferred_element_type=jnp.float32)
    u = lax.dot_general(xb, wu_ref[0], (((1,), (1,)), ((), ())),
                        preferred_element_type=jnp.float32)
    a = ((g * jax.nn.sigmoid(g)) * u * w_ref[...]).astype(jnp.bfloat16)
    ys_ref[...] = lax.dot_general(a, wd_ref[0], (((1,), (1,)), ((), ())),
                                  preferred_element_type=jnp.float32)


def _grouped_mlp(tile_eid, xs, sorted_w, wg, wu, wd):
    grid_spec = pltpu.PrefetchScalarGridSpec(
        num_scalar_prefetch=1,
        grid=(NT,),
        in_specs=[
            pl.BlockSpec((TG, H), lambda i, eid: (i, 0)),
            pl.BlockSpec((TG, 1), lambda i, eid: (i, 0)),
            pl.BlockSpec((1, FF, H), lambda i, eid: (eid[i], 0, 0)),
            pl.BlockSpec((1, FF, H), lambda i, eid: (eid[i], 0, 0)),
            pl.BlockSpec((1, H, FF), lambda i, eid: (eid[i], 0, 0)),
        ],
        out_specs=pl.BlockSpec((TG, H), lambda i, eid: (i, 0)),
    )
    return pl.pallas_call(
        _grouped_mlp_body,
        grid_spec=grid_spec,
        out_shape=jax.ShapeDtypeStruct((NPAD, H), jnp.float32),
    )(tile_eid, xs, sorted_w, wg, wu, wd)


SC_NC, SC_NS = 2, 16
NW = SC_NC * SC_NS
GR = 128


@functools.lru_cache(maxsize=None)
def _sc_mesh():
    return plsc.VectorSubcoreMesh(core_axis_name="c", subcore_axis_name="s")


@functools.lru_cache(maxsize=None)
def _sc_gather_kernel(n_out):
    def body(tab_hbm, idx_hbm, out_hbm, idx_v, rows_v, sem):
        wid = lax.axis_index("s") * SC_NC + lax.axis_index("c")
        rows_per_w = n_out // NW
        base_w = wid * rows_per_w

        def chunk(i, _):
            base = base_w + i * GR
            pltpu.sync_copy(idx_hbm.at[pl.ds(base, GR)], idx_v)
            pltpu.async_copy(tab_hbm.at[idx_v], rows_v, sem).wait()
            pltpu.sync_copy(rows_v, out_hbm.at[pl.ds(base, GR)])
            return _

        lax.fori_loop(0, rows_per_w // GR, chunk, 0)

    return pl.kernel(
        body,
        out_type=jax.ShapeDtypeStruct((n_out, H), jnp.float32),
        mesh=_sc_mesh(),
        scratch_types=[
            pltpu.VMEM((GR,), jnp.int32),
            pltpu.VMEM((GR, H), jnp.float32),
            pltpu.SemaphoreType.DMA,
        ],
    )


def _combine_body(g01_ref, g1_ref, sh_ref, y_ref):
    y_ref[...] = (g01_ref[...].astype(jnp.float32)
                  + g1_ref[...].astype(jnp.float32)
                  + sh_ref[...].astype(jnp.float32))


def _combine(g01, sh):
    nb = T // TM
    return pl.pallas_call(
        _combine_body,
        grid=(nb,),
        in_specs=[
            pl.BlockSpec((TM, H), lambda i: (i, 0)),
            pl.BlockSpec((TM, H), lambda i, nb=nb: (i + nb, 0)),
            pl.BlockSpec((TM, H), lambda i: (i, 0)),
        ],
        out_specs=pl.BlockSpec((TM, H), lambda i: (i, 0)),
        out_shape=jax.ShapeDtypeStruct((T, H), jnp.float32),
    )(g01, g01, sh)


def _route(idx, wts):
    eid = idx.reshape(N)
    wf = wts.reshape(N)
    CH = 128
    oh = (eid[:, None] == jnp.arange(E, dtype=jnp.int32)[None, :]
          ).astype(jnp.float32).reshape(N // CH, CH, E)
    tril = jnp.tril(jnp.ones((CH, CH), jnp.float32))
    within = jnp.einsum('rc,bce->bre', tril, oh,
                        preferred_element_type=jnp.float32)
    blk_tot = within[:, -1, :]
    blk_pre = jnp.cumsum(blk_tot, axis=0) - blk_tot
    ranks = (within + blk_pre[:, None, :]).reshape(N, E)
    counts = (blk_pre[-1] + blk_tot[-1]).astype(jnp.int32)
    padded = ((counts + TG - 1) // TG) * TG
    pcum = jnp.cumsum(padded)
    base = pcum - padded
    rank_n = jnp.take_along_axis(ranks, eid[:, None], axis=1)[:, 0]
    pos = base[eid] + rank_n.astype(jnp.int32) - 1
    sorted_tok = (jnp.arange(NPAD, dtype=jnp.int32) % T).at[pos].set(
        jnp.arange(N, dtype=jnp.int32) // TOPK)
    sorted_w = jnp.zeros((NPAD,), jnp.float32).at[pos].set(wf)
    tile_start = jnp.arange(NT, dtype=jnp.int32) * TG
    tile_eid = jnp.minimum(
        jnp.sum((tile_start[:, None] >= pcum[None, :]).astype(jnp.int32),
                axis=1), E - 1).astype(jnp.int32)
    return pos, sorted_tok, sorted_w, tile_eid


@jax.jit
def _moe(x, gw, wg, wu, wd, s1, s2, s3):
    idx, wts, shared = _gate_shared(x, gw, s1, s2, s3)
    pos, sorted_tok, sorted_w, tile_eid = _route(idx, wts)
    xs = _sc_gather_kernel(NPAD)(x, sorted_tok)
    ys = _grouped_mlp(tile_eid, xs, sorted_w[:, None], wg, wu, wd)
    pos2 = pos.reshape(T, TOPK)
    pos_cat = jnp.concatenate([pos2[:, 0], pos2[:, 1]])
    g01 = _sc_gather_kernel(N)(ys, pos_cat)
    return _combine(g01, shared)


def kernel(hidden_states, gate_weight, Wg, Wu, Wd, sWg, sWu, sWd):
    x = hidden_states.reshape(T, H)
    wg = Wg.astype(jnp.bfloat16)
    wu = Wu.astype(jnp.bfloat16)
    wd = Wd.astype(jnp.bfloat16)
    s1 = sWg.astype(jnp.bfloat16)
    s2 = sWu.astype(jnp.bfloat16)
    s3 = sWd.T.astype(jnp.bfloat16)
    y = _moe(x, gate_weight, wg, wu, wd, s1, s2, s3)
    return y.reshape(B, S, H)


# speedup vs baseline: 3.7749x; 1.1430x over previous
import functools

import jax
import jax.numpy as jnp
from jax import lax
from jax.experimental import pallas as pl
from jax.experimental.pallas import tpu as pltpu
from jax.experimental.pallas import tpu_sc as plsc

B, S, H = 2, 4096, 768
E, TOPK, FF = 16, 2, 384
SFF = 384 * 2
T = B * S
N = T * TOPK
TM = 256
TG = 256
NPAD = N + E * TG
NT = NPAD // TG


def _gate_shared_body(x_ref, gw_ref, s1_ref, s2_ref, s3_ref,
                      idx_ref, w_ref, sh_ref):
    x = x_ref[...]
    logits = lax.dot_general(x, gw_ref[...], (((1,), (1,)), ((), ())),
                             preferred_element_type=jnp.float32)
    m = jnp.max(logits, axis=-1, keepdims=True)
    p = jnp.exp(logits - m)
    s = p / jnp.sum(p, axis=-1, keepdims=True)
    iota = lax.broadcasted_iota(jnp.int32, (TM, E), 1)
    m1 = jnp.max(s, axis=-1, keepdims=True)
    i1 = jnp.min(jnp.where(s == m1, iota, E), axis=-1, keepdims=True)
    oh1 = iota == i1
    s2 = jnp.where(oh1, -1.0, s)
    m2 = jnp.max(s2, axis=-1, keepdims=True)
    i2 = jnp.min(jnp.where(s2 == m2, iota, E), axis=-1, keepdims=True)
    denom = m1 + m2 + 1e-20
    idx_ref[...] = jnp.concatenate([i1, i2], axis=1)
    w_ref[...] = jnp.concatenate([m1 / denom, m2 / denom], axis=1)
    xb = x.astype(jnp.bfloat16)
    g = lax.dot_general(xb, s1_ref[...], (((1,), (1,)), ((), ())),
                        preferred_element_type=jnp.float32)
    u = lax.dot_general(xb, s2_ref[...], (((1,), (1,)), ((), ())),
                        preferred_element_type=jnp.float32)
    a = ((g * jax.nn.sigmoid(g)) * u).astype(jnp.bfloat16)
    sh_ref[...] = lax.dot_general(a, s3_ref[...], (((1,), (0,)), ((), ())),
                                  preferred_element_type=jnp.float32)


def _gate_shared(x, gw, s1, s2, s3):
    return pl.pallas_call(
        _gate_shared_body,
        grid=(T // TM,),
        in_specs=[
            pl.BlockSpec((TM, H), lambda i: (i, 0)),
            pl.BlockSpec((E, H), lambda i: (0, 0)),
            pl.BlockSpec((SFF, H), lambda i: (0, 0)),
            pl.BlockSpec((SFF, H), lambda i: (0, 0)),
            pl.BlockSpec((SFF, H), lambda i: (0, 0)),
        ],
        out_specs=[
            pl.BlockSpec((TM, TOPK), lambda i: (i, 0)),
            pl.BlockSpec((TM, TOPK), lambda i: (i, 0)),
            pl.BlockSpec((TM, H), lambda i: (i, 0)),
        ],
        out_shape=[
            jax.ShapeDtypeStruct((T, TOPK), jnp.int32),
            jax.ShapeDtypeStruct((T, TOPK), jnp.float32),
            jax.ShapeDtypeStruct((T, H), jnp.float32),
        ],
    )(x, gw, s1, s2, s3)


def _grouped_mlp_body(eid_ref, xs_ref, wg_ref, wu_ref, wd_ref, ys_ref):
    xb = xs_ref[...].astype(jnp.bfloat16)
    g = lax.dot_general(xb, wg_ref[0], (((1,), (1,)), ((), ())),
                        preferred_element_type=jnp.float32)
    u = lax.dot_general(xb, wu_ref[0], (((1,), (1,)), ((), ())),
                        preferred_element_type=jnp.float32)
    a = ((g * jax.nn.sigmoid(g)) * u).astype(jnp.bfloat16)
    ys_ref[...] = lax.dot_general(a, wd_ref[0], (((1,), (1,)), ((), ())),
                                  preferred_element_type=jnp.float32)


def _grouped_mlp(tile_eid, xs, wg, wu, wd):
    grid_spec = pltpu.PrefetchScalarGridSpec(
        num_scalar_prefetch=1,
        grid=(NT,),
        in_specs=[
            pl.BlockSpec((TG, H), lambda i, eid: (i, 0)),
            pl.BlockSpec((1, FF, H), lambda i, eid: (eid[i], 0, 0)),
            pl.BlockSpec((1, FF, H), lambda i, eid: (eid[i], 0, 0)),
            pl.BlockSpec((1, H, FF), lambda i, eid: (eid[i], 0, 0)),
        ],
        out_specs=pl.BlockSpec((TG, H), lambda i, eid: (i, 0)),
    )
    return pl.pallas_call(
        _grouped_mlp_body,
        grid_spec=grid_spec,
        out_shape=jax.ShapeDtypeStruct((NPAD, H), jnp.float32),
    )(tile_eid, xs, wg, wu, wd)


SC_NC, SC_NS = 2, 16
NW = SC_NC * SC_NS
GR = 128


@functools.lru_cache(maxsize=None)
def _sc_mesh():
    return plsc.VectorSubcoreMesh(core_axis_name="c", subcore_axis_name="s")


@functools.lru_cache(maxsize=None)
def _sc_gather_kernel(n_out):
    def body(tab_hbm, idx_hbm, out_hbm, idx_v, rows_v, sem):
        wid = lax.axis_index("s") * SC_NC + lax.axis_index("c")
        rows_per_w = n_out // NW
        base_w = wid * rows_per_w

        def chunk(i, _):
            base = base_w + i * GR
            pltpu.sync_copy(idx_hbm.at[pl.ds(base, GR)], idx_v)
            pltpu.async_copy(tab_hbm.at[idx_v], rows_v, sem).wait()
            pltpu.sync_copy(rows_v, out_hbm.at[pl.ds(base, GR)])
            return _

        lax.fori_loop(0, rows_per_w // GR, chunk, 0)

    return pl.kernel(
        body,
        out_type=jax.ShapeDtypeStruct((n_out, H), jnp.float32),
        mesh=_sc_mesh(),
        scratch_types=[
            pltpu.VMEM((GR,), jnp.int32),
            pltpu.VMEM((GR, H), jnp.float32),
            pltpu.SemaphoreType.DMA,
        ],
    )


def _combine_body(g0_ref, g1_ref, sh_ref, w_ref, y_ref):
    w = w_ref[...]
    y_ref[...] = (g0_ref[...] * w[:, 0:1] + g1_ref[...] * w[:, 1:2]
                  + sh_ref[...])


def _combine(g01, sh, wts):
    nb = T // TM
    return pl.pallas_call(
        _combine_body,
        grid=(nb,),
        in_specs=[
            pl.BlockSpec((TM, H), lambda i: (i, 0)),
            pl.BlockSpec((TM, H), lambda i, nb=nb: (i + nb, 0)),
            pl.BlockSpec((TM, H), lambda i: (i, 0)),
            pl.BlockSpec((TM, TOPK), lambda i: (i, 0)),
        ],
        out_specs=pl.BlockSpec((TM, H), lambda i: (i, 0)),
        out_shape=jax.ShapeDtypeStruct((T, H), jnp.float32),
    )(g01, g01, sh, wts)


def _route(idx):
    eid = idx.reshape(N)
    CH = 128
    oh = (eid[:, None] == jnp.arange(E, dtype=jnp.int32)[None, :]
          ).astype(jnp.float32).reshape(N // CH, CH, E)
    tril = jnp.tril(jnp.ones((CH, CH), jnp.float32))
    within = jnp.einsum('rc,bce->bre', tril, oh,
                        preferred_element_type=jnp.float32)
    blk_tot = within[:, -1, :]
    blk_pre = jnp.cumsum(blk_tot, axis=0) - blk_tot
    ranks = (within + blk_pre[:, None, :]).reshape(N, E)
    counts = (blk_pre[-1] + blk_tot[-1]).astype(jnp.int32)
    padded = ((counts + TG - 1) // TG) * TG
    pcum = jnp.cumsum(padded)
    base = pcum - padded
    rank_n = jnp.take_along_axis(ranks, eid[:, None], axis=1)[:, 0]
    pos = base[eid] + rank_n.astype(jnp.int32) - 1
    sorted_tok = (jnp.arange(NPAD, dtype=jnp.int32) % T).at[pos].set(
        jnp.arange(N, dtype=jnp.int32) // TOPK)
    tile_start = jnp.arange(NT, dtype=jnp.int32) * TG
    tile_eid = jnp.minimum(
        jnp.sum((tile_start[:, None] >= pcum[None, :]).astype(jnp.int32),
                axis=1), E - 1).astype(jnp.int32)
    return pos, sorted_tok, tile_eid


@jax.jit
def _moe(x, gw, wg, wu, wd, s1, s2, s3):
    idx, wts, shared = _gate_shared(x, gw, s1, s2, s3)
    pos, sorted_tok, tile_eid = _route(idx)
    xs = _sc_gather_kernel(NPAD)(x, sorted_tok)
    ys = _grouped_mlp(tile_eid, xs, wg, wu, wd)
    pos2 = pos.reshape(T, TOPK)
    pos_cat = jnp.concatenate([pos2[:, 0], pos2[:, 1]])
    g01 = _sc_gather_kernel(N)(ys, pos_cat)
    return _combine(g01, shared, wts)


def kernel(hidden_states, gate_weight, Wg, Wu, Wd, sWg, sWu, sWd):
    x = hidden_states.reshape(T, H)
    wg = Wg.astype(jnp.bfloat16)
    wu = Wu.astype(jnp.bfloat16)
    wd = Wd.astype(jnp.bfloat16)
    s1 = sWg.astype(jnp.bfloat16)
    s2 = sWu.astype(jnp.bfloat16)
    s3 = sWd.T.astype(jnp.bfloat16)
    y = _moe(x, gate_weight, wg, wu, wd, s1, s2, s3)
    return y.reshape(B, S, H)
